# Initial kernel scaffold; baseline (speedup 1.0000x reference)
import jax
import jax.numpy as jnp
from jax.experimental import pallas as pl


def kernel(item, intent, mask, b_seq, b_seq2, type_cnt, W_item, W_intent):
    raise NotImplementedError("write your pallas kernel here")


# speedup vs baseline: 3.3920x; 3.3920x over previous
import functools
import math

import jax
import jax.numpy as jnp
from jax import lax
from jax.experimental import pallas as pl
from jax.experimental.pallas import tpu as pltpu

N_H = 12
N_I = 4
N_B = 4
D_MODEL = 768
D_K = 64
NB = 24
MAX_SEQ_LEN = 2048
BS = 2
MAXLEN = 2048
N_IS = N_I * (N_B + 1)
N_TYPES_ITEM = N_B + 2
N_TYPES_INT = N_B + 1
NBLK = 512
KMAX = 11

def _dot(a, b):
    return jnp.dot(a.astype(jnp.bfloat16), b.astype(jnp.bfloat16),
                   preferred_element_type=jnp.float32)


def _get_cn_vec(x):
    inner = jnp.floor(
        jnp.log(jnp.maximum(4.0 * x / NB, 1e-20))
        / math.log(4.0 * MAX_SEQ_LEN / NB) * (NB / 4.0))
    f1 = NB / 4.0 + inner
    ub = NB / 2.0 - 1.0
    alt = (f1 + ub - jnp.abs(f1 - ub)) * 0.5
    return jnp.where(x < NB / 4.0, x, alt)


def _proj_body(item_ref, bseq_ref, wk_ref, wv_ref,
               kbs_ref, vbs_ref, kba_ref, vba_ref):
    x = item_ref[0]
    bt = bseq_ref[0]
    kba = _dot(x, wk_ref[N_TYPES_ITEM - 1])
    vba = _dot(x, wv_ref[N_TYPES_ITEM - 1])
    acc_k = jnp.where(bt == float(N_TYPES_ITEM - 1), kba, 0.0)
    acc_v = jnp.where(bt == float(N_TYPES_ITEM - 1), vba, 0.0)
    zero = jnp.zeros_like(x)
    for t in range(N_TYPES_ITEM - 1):
        xm = jnp.where(bt == float(t), x, zero)
        acc_k = acc_k + _dot(xm, wk_ref[t])
        acc_v = acc_v + _dot(xm, wv_ref[t])
    kbs_ref[0] = acc_k.astype(jnp.bfloat16)
    vbs_ref[0] = acc_v.astype(jnp.bfloat16)
    kba_ref[0] = kba.astype(jnp.bfloat16)
    vba_ref[0] = vba.astype(jnp.bfloat16)


def _attend(q, k, v, m, k_col):
    s = lax.dot_general(q.astype(jnp.bfloat16), k.astype(jnp.bfloat16),
                        (((1,), (1,)), ((), ())),
                        preferred_element_type=jnp.float32) * (1.0 / math.sqrt(D_K))
    s = jnp.where(m == 0, -1e30, s)
    mx = jnp.max(s, axis=-1, keepdims=True)
    e = jnp.exp(s - mx)
    denom = jnp.sum(e, axis=-1, keepdims=True)
    thr = jnp.full_like(mx, jnp.inf)
    cur = s
    for j in range(KMAX):
        mj = jnp.max(cur, axis=-1, keepdims=True)
        thr = jnp.where(k_col == float(j + 1), mj, thr)
        cur = jnp.where(cur >= mj, -jnp.inf, cur)
    p = jnp.where(s >= thr, e, 0.0) / denom
    return _dot(p, v)


def _attn_body(intent_ref, bseq2_ref, wq_ref, kbs_ref, vbs_ref, kba_ref,
               vba_ref, mask_ref, cntrep_ref, cntrow_ref, out_ref, q_scr):
    xi = intent_ref[0]
    bt2 = bseq2_ref[0]
    q = jnp.zeros((N_IS, D_MODEL), jnp.float32)
    zero = jnp.zeros_like(xi)
    for t in range(N_TYPES_INT):
        q = q + _dot(jnp.where(bt2 == float(t), xi, zero), wq_ref[t])
    q_scr[0:N_IS, :] = q.astype(jnp.bfloat16)

    counts16 = cntrep_ref[0]
    counts4 = cntrow_ref[0]
    k_bs = _get_cn_vec(counts16)
    k_ba = _get_cn_vec(jnp.sum(counts4, axis=1, keepdims=True))
    n_bs = N_B * N_I
    mask_bs = mask_ref[0, 0:n_bs, :]
    mask_ba = mask_ref[0, n_bs:N_IS, :]

    for h in range(N_H):
        sl = slice(D_K * h, D_K * (h + 1))
        q_bs = q_scr[0:n_bs, sl]
        q_ba = q_scr[n_bs:N_IS, sl]
        o_bs = _attend(q_bs, kbs_ref[0, :, sl], vbs_ref[0, :, sl],
                       mask_bs, k_bs)
        o_ba = _attend(q_ba, kba_ref[0, :, sl], vba_ref[0, :, sl],
                       mask_ba, k_ba)
        out_ref[0, h, 0:n_bs, :] = o_bs
        out_ref[0, h, n_bs:N_IS, :] = o_ba


def kernel(item, intent, mask, b_seq, b_seq2, type_cnt, W_item, W_intent):
    bs = item.shape[0]
    wk = W_item[0].reshape(N_TYPES_ITEM, D_MODEL, N_H * D_K).astype(jnp.bfloat16)
    wv = W_item[1].reshape(N_TYPES_ITEM, D_MODEL, N_H * D_K).astype(jnp.bfloat16)
    wq = W_intent[0].reshape(N_TYPES_INT, D_MODEL, N_H * D_K).astype(jnp.bfloat16)
    item = item.astype(jnp.bfloat16)
    intent = intent.astype(jnp.bfloat16)
    bseq_f = b_seq.astype(jnp.float32)[..., None]
    bseq2_f = b_seq2.astype(jnp.float32)[..., None]
    mask2 = mask.reshape(bs, N_IS, MAXLEN)
    cnt_rep = jnp.repeat(type_cnt.astype(jnp.float32), N_I, axis=1)[..., None]
    cnt_row = type_cnt.astype(jnp.float32)[:, None, :]

    nblks = MAXLEN // NBLK
    hk = N_H * D_K
    kv_shape = jax.ShapeDtypeStruct((bs, MAXLEN, hk), jnp.bfloat16)
    kbs, vbs, kba, vba = pl.pallas_call(
        _proj_body,
        grid=(bs, nblks),
        in_specs=[
            pl.BlockSpec((1, NBLK, D_MODEL), lambda b, n: (b, n, 0)),
            pl.BlockSpec((1, NBLK, 1), lambda b, n: (b, n, 0)),
            pl.BlockSpec((N_TYPES_ITEM, D_MODEL, hk), lambda b, n: (0, 0, 0)),
            pl.BlockSpec((N_TYPES_ITEM, D_MODEL, hk), lambda b, n: (0, 0, 0)),
        ],
        out_specs=[
            pl.BlockSpec((1, NBLK, hk), lambda b, n: (b, n, 0)),
            pl.BlockSpec((1, NBLK, hk), lambda b, n: (b, n, 0)),
            pl.BlockSpec((1, NBLK, hk), lambda b, n: (b, n, 0)),
            pl.BlockSpec((1, NBLK, hk), lambda b, n: (b, n, 0)),
        ],
        out_shape=[kv_shape, kv_shape, kv_shape, kv_shape],
    )(item, bseq_f, wk, wv)

    out = pl.pallas_call(
        _attn_body,
        grid=(bs,),
        in_specs=[
            pl.BlockSpec((1, N_IS, D_MODEL), lambda b: (0, 0, 0)),
            pl.BlockSpec((1, N_IS, 1), lambda b: (b, 0, 0)),
            pl.BlockSpec((N_TYPES_INT, D_MODEL, hk), lambda b: (0, 0, 0)),
            pl.BlockSpec((1, MAXLEN, hk), lambda b: (b, 0, 0)),
            pl.BlockSpec((1, MAXLEN, hk), lambda b: (b, 0, 0)),
            pl.BlockSpec((1, MAXLEN, hk), lambda b: (b, 0, 0)),
            pl.BlockSpec((1, MAXLEN, hk), lambda b: (b, 0, 0)),
            pl.BlockSpec((1, N_IS, MAXLEN), lambda b: (b, 0, 0)),
            pl.BlockSpec((1, N_B * N_I, 1), lambda b: (b, 0, 0)),
            pl.BlockSpec((1, 1, N_B), lambda b: (b, 0, 0)),
        ],
        out_specs=pl.BlockSpec((1, N_H, N_IS, D_K), lambda b: (b, 0, 0, 0)),
        out_shape=jax.ShapeDtypeStruct((bs, N_H, N_IS, D_K), jnp.float32),
        scratch_shapes=[pltpu.VMEM((32, D_MODEL), jnp.bfloat16)],
        compiler_params=pltpu.CompilerParams(vmem_limit_bytes=100 * 2**20),
    )(intent, bseq2_f, wq, kbs, vbs, kba, vba, mask2, cnt_rep, cnt_row)

    return jnp.transpose(out, (0, 2, 1, 3)).reshape(bs, N_IS, N_H * D_K)


# speedup vs baseline: 4.1045x; 1.2100x over previous
import math

import jax
import jax.numpy as jnp
from jax import lax
from jax.experimental import pallas as pl
from jax.experimental.pallas import tpu as pltpu

N_H = 12
N_I = 4
N_B = 4
D_MODEL = 768
D_K = 64
NB = 24
MAX_SEQ_LEN = 2048
MAXLEN = 2048
N_IS = N_I * (N_B + 1)
N_BS = N_B * N_I
N_TYPES_ITEM = N_B + 2
N_TYPES_INT = N_B + 1
NBLK = 512
KMAX = 11


def _dot(a, b):
    return jnp.dot(a.astype(jnp.bfloat16), b.astype(jnp.bfloat16),
                   preferred_element_type=jnp.float32)


def _get_cn_vec(x):
    inner = jnp.floor(
        jnp.log(jnp.maximum(4.0 * x / NB, 1e-20))
        / math.log(4.0 * MAX_SEQ_LEN / NB) * (NB / 4.0))
    f1 = NB / 4.0 + inner
    ub = NB / 2.0 - 1.0
    alt = (f1 + ub - jnp.abs(f1 - ub)) * 0.5
    return jnp.where(x < NB / 4.0, x, alt)


def _expand_types(x, type_col, n_types):
    zero = jnp.zeros_like(x)
    return jnp.concatenate(
        [jnp.where(type_col == float(t), x, zero) for t in range(n_types)],
        axis=1)


def _proj_body(item_ref, bseq_ref, wk_ref, wv_ref,
               kbs_ref, vbs_ref, kba_ref, vba_ref):
    x = item_ref[0].astype(jnp.bfloat16)
    bt = bseq_ref[0]
    x6 = _expand_types(x, bt, N_TYPES_ITEM)
    wk_flat = wk_ref[...].reshape(N_TYPES_ITEM * D_MODEL, N_H * D_K)
    wv_flat = wv_ref[...].reshape(N_TYPES_ITEM * D_MODEL, N_H * D_K)
    kbs_ref[0] = _dot(x6, wk_flat).astype(jnp.bfloat16)
    vbs_ref[0] = _dot(x6, wv_flat).astype(jnp.bfloat16)
    kba_ref[0] = _dot(x, wk_ref[N_TYPES_ITEM - 1]).astype(jnp.bfloat16)
    vba_ref[0] = _dot(x, wv_ref[N_TYPES_ITEM - 1]).astype(jnp.bfloat16)


def _softmax_topk(s_raw, m, k_row, nrep):
    mrep = jnp.concatenate([m] * N_H, axis=1)
    s = jnp.where(mrep == 0, -1e30, s_raw * (1.0 / math.sqrt(D_K)))
    mx = jnp.max(s, axis=0, keepdims=True)
    e = jnp.exp(s - mx)
    den = jnp.sum(e, axis=0, keepdims=True)
    if k_row.shape[1] > 1:
        k_rep = jnp.concatenate([k_row] * N_H, axis=1)
    else:
        k_rep = k_row
    thr = jnp.full((1, s.shape[1]), jnp.inf, jnp.float32)
    cur = s
    for j in range(KMAX):
        mj = jnp.max(cur, axis=0, keepdims=True)
        thr = jnp.where(k_rep == float(j + 1), mj, thr)
        cur = jnp.where(cur >= mj, -jnp.inf, cur)
    return jnp.where(s >= thr, e, 0.0) / den


def _attn_body(intent_ref, bseq2_ref, wq_ref, kbs_ref, vbs_ref, kba_ref,
               vba_ref, maskt_ref, cntrep_ref, cntrow_ref, out_ref):
    xi = intent_ref[0].astype(jnp.bfloat16)
    bt2 = bseq2_ref[0]
    xi5 = _expand_types(xi, bt2, N_TYPES_INT)
    wq_flat = wq_ref[...].reshape(N_TYPES_INT * D_MODEL, N_H * D_K)
    q = _dot(xi5, wq_flat)
    qt = jnp.transpose(q).astype(jnp.bfloat16)

    rowh = lax.broadcasted_iota(jnp.int32, (D_MODEL, 1), 0) // D_K
    qt_bs = qt[:, 0:N_BS]
    qt_ba = qt[:, N_BS:N_IS]
    zb = jnp.zeros_like(qt_bs)
    za = jnp.zeros_like(qt_ba)
    qd_bs = jnp.concatenate(
        [jnp.where(rowh == h, qt_bs, zb) for h in range(N_H)], axis=1)
    qd_ba = jnp.concatenate(
        [jnp.where(rowh == h, qt_ba, za) for h in range(N_H)], axis=1)

    maskt = maskt_ref[0]
    k_bs = _get_cn_vec(cntrep_ref[0])
    k_ba = _get_cn_vec(jnp.sum(cntrow_ref[0], axis=1, keepdims=True))

    s_bs = _dot(kbs_ref[0], qd_bs)
    p_bs = _softmax_topk(s_bs, maskt[:, 0:N_BS], k_bs, N_H)
    res_bs = _dot(jnp.transpose(p_bs), vbs_ref[0])

    s_ba = _dot(kba_ref[0], qd_ba)
    p_ba = _softmax_topk(s_ba, maskt[:, N_BS:N_IS], k_ba, N_H)
    res_ba = _dot(jnp.transpose(p_ba), vba_ref[0])

    for h in range(N_H):
        cols = slice(D_K * h, D_K * (h + 1))
        out_ref[0, h, 0:N_BS, :] = res_bs[N_BS * h:N_BS * (h + 1), cols]
        out_ref[0, h, N_BS:N_IS, :] = res_ba[N_I * h:N_I * (h + 1), cols]


def kernel(item, intent, mask, b_seq, b_seq2, type_cnt, W_item, W_intent):
    bs = item.shape[0]
    hk = N_H * D_K
    wk = W_item[0].reshape(N_TYPES_ITEM, D_MODEL, hk).astype(jnp.bfloat16)
    wv = W_item[1].reshape(N_TYPES_ITEM, D_MODEL, hk).astype(jnp.bfloat16)
    wq = W_intent[0].reshape(N_TYPES_INT, D_MODEL, hk).astype(jnp.bfloat16)
    bseq_f = b_seq.astype(jnp.float32)[..., None]
    bseq2_f = b_seq2.astype(jnp.float32)[..., None]
    mask_t = jnp.transpose(mask.reshape(bs, N_IS, MAXLEN), (0, 2, 1))
    cnt_rep = jnp.repeat(type_cnt.astype(jnp.float32), N_I, axis=1)[:, None, :]
    cnt_row = type_cnt.astype(jnp.float32)[:, None, :]

    nblks = MAXLEN // NBLK
    kv_shape = jax.ShapeDtypeStruct((bs, MAXLEN, hk), jnp.bfloat16)
    kbs, vbs, kba, vba = pl.pallas_call(
        _proj_body,
        grid=(bs, nblks),
        in_specs=[
            pl.BlockSpec((1, NBLK, D_MODEL), lambda b, n: (b, n, 0)),
            pl.BlockSpec((1, NBLK, 1), lambda b, n: (b, n, 0)),
            pl.BlockSpec((N_TYPES_ITEM, D_MODEL, hk), lambda b, n: (0, 0, 0)),
            pl.BlockSpec((N_TYPES_ITEM, D_MODEL, hk), lambda b, n: (0, 0, 0)),
        ],
        out_specs=[
            pl.BlockSpec((1, NBLK, hk), lambda b, n: (b, n, 0)),
            pl.BlockSpec((1, NBLK, hk), lambda b, n: (b, n, 0)),
            pl.BlockSpec((1, NBLK, hk), lambda b, n: (b, n, 0)),
            pl.BlockSpec((1, NBLK, hk), lambda b, n: (b, n, 0)),
        ],
        out_shape=[kv_shape, kv_shape, kv_shape, kv_shape],
    )(item, bseq_f, wk, wv)

    out = pl.pallas_call(
        _attn_body,
        grid=(bs,),
        in_specs=[
            pl.BlockSpec((1, N_IS, D_MODEL), lambda b: (0, 0, 0)),
            pl.BlockSpec((1, N_IS, 1), lambda b: (b, 0, 0)),
            pl.BlockSpec((N_TYPES_INT, D_MODEL, hk), lambda b: (0, 0, 0)),
            pl.BlockSpec((1, MAXLEN, hk), lambda b: (b, 0, 0)),
            pl.BlockSpec((1, MAXLEN, hk), lambda b: (b, 0, 0)),
            pl.BlockSpec((1, MAXLEN, hk), lambda b: (b, 0, 0)),
            pl.BlockSpec((1, MAXLEN, hk), lambda b: (b, 0, 0)),
            pl.BlockSpec((1, MAXLEN, N_IS), lambda b: (b, 0, 0)),
            pl.BlockSpec((1, 1, N_BS), lambda b: (b, 0, 0)),
            pl.BlockSpec((1, 1, N_B), lambda b: (b, 0, 0)),
        ],
        out_specs=pl.BlockSpec((1, N_H, N_IS, D_K), lambda b: (b, 0, 0, 0)),
        out_shape=jax.ShapeDtypeStruct((bs, N_H, N_IS, D_K), jnp.float32),
        compiler_params=pltpu.CompilerParams(vmem_limit_bytes=100 * 2**20),
    )(intent, bseq2_f, wq, kbs, vbs, kba, vba, mask_t, cnt_rep, cnt_row)

    return jnp.transpose(out, (0, 2, 1, 3)).reshape(bs, N_IS, hk)
